# baseline (speedup 1.0000x reference)
import functools

import jax
import jax.numpy as jnp
from jax import lax
from jax.experimental import pallas as pl
from jax.experimental.pallas import tpu as pltpu
from jax.experimental.pallas import tpu_sc as plsc

NUM_CORES = 2
NUM_SUBCORES = 16
LANES = 16
NUM_WORKERS = NUM_CORES * NUM_SUBCORES

NUM_USERS = 100000
NUM_ITEMS = 1000000
BATCH = 16384
EMBED_DIM = 32
B_PER_W = BATCH // NUM_WORKERS
GATHER_CHUNK = 128
FLAT_PER_W = B_PER_W * EMBED_DIM


def _sc_body(uidx_hbm, iidx_hbm, uembT_hbm, iembT_hbm, out_hbm,
             uidx_v, iidx_v, ufidx, ifidx, ubuf, ibuf, out_v, sem_u, sem_i):
    wid = lax.axis_index("s") * NUM_CORES + lax.axis_index("c")
    base = wid * B_PER_W

    pltpu.sync_copy(uidx_hbm.at[pl.ds(base, B_PER_W)], uidx_v)
    pltpu.sync_copy(iidx_hbm.at[pl.ds(base, B_PER_W)], iidx_v)

    def build(g, carry):
        off = pl.multiple_of(g * LANES, LANES)
        uvec = uidx_v[pl.ds(off, LANES)]
        ivec = iidx_v[pl.ds(off, LANES)]
        for d in range(EMBED_DIM):
            ufidx[pl.ds(d * B_PER_W + off, LANES)] = uvec + d * NUM_USERS
            ifidx[pl.ds(d * B_PER_W + off, LANES)] = ivec + d * NUM_ITEMS
        return carry

    lax.fori_loop(0, B_PER_W // LANES, build, 0, unroll=False)

    uflat = uembT_hbm
    iflat = iembT_hbm
    copies = []
    for c in range(FLAT_PER_W // GATHER_CHUNK):
        sl = pl.ds(c * GATHER_CHUNK, GATHER_CHUNK)
        copies.append(pltpu.async_copy(uflat.at[ufidx.at[sl]], ubuf.at[sl], sem_u))
        copies.append(pltpu.async_copy(iflat.at[ifidx.at[sl]], ibuf.at[sl], sem_i))
    for c in copies:
        c.wait()

    def group(g, carry):
        b0 = pl.multiple_of(g * LANES, LANES)
        acc = jnp.zeros((LANES,), jnp.float32)
        for d in range(EMBED_DIM):
            acc = acc + (ubuf[pl.ds(d * B_PER_W + b0, LANES)]
                         * ibuf[pl.ds(d * B_PER_W + b0, LANES)])
        out_v[pl.ds(b0, LANES)] = acc
        return carry

    lax.fori_loop(0, B_PER_W // LANES, group, 0, unroll=False)

    pltpu.sync_copy(out_v, out_hbm.at[pl.ds(base, B_PER_W)])


def kernel(user_indices, item_indices, user_emb, item_emb):
    mesh = plsc.VectorSubcoreMesh(
        core_axis_name="c", subcore_axis_name="s",
        num_cores=NUM_CORES, num_subcores=NUM_SUBCORES)
    run = pl.kernel(
        _sc_body,
        out_type=jax.ShapeDtypeStruct((BATCH,), jnp.float32),
        mesh=mesh,
        compiler_params=pltpu.CompilerParams(
            needs_layout_passes=False, use_tc_tiling_on_sc=False),
        scratch_types=[
            pltpu.VMEM((B_PER_W,), jnp.int32),
            pltpu.VMEM((B_PER_W,), jnp.int32),
            pltpu.VMEM((FLAT_PER_W,), jnp.int32),
            pltpu.VMEM((FLAT_PER_W,), jnp.int32),
            pltpu.VMEM((FLAT_PER_W,), jnp.float32),
            pltpu.VMEM((FLAT_PER_W,), jnp.float32),
            pltpu.VMEM((B_PER_W,), jnp.float32),
            pltpu.SemaphoreType.DMA,
            pltpu.SemaphoreType.DMA,
        ],
    )
    return run(user_indices.astype(jnp.int32),
               item_indices.astype(jnp.int32),
               user_emb.T.reshape(-1), item_emb.T.reshape(-1))


# speedup vs baseline: 11.9439x; 11.9439x over previous
import functools

import jax
import jax.numpy as jnp
from jax import lax
from jax.experimental import pallas as pl
from jax.experimental.pallas import tpu as pltpu
from jax.experimental.pallas import tpu_sc as plsc

NUM_CORES = 2
NUM_SUBCORES = 16
LANES = 16
NUM_WORKERS = NUM_CORES * NUM_SUBCORES

NUM_USERS = 100000
NUM_ITEMS = 1000000
BATCH = 16384
EMBED_DIM = 32
B_PER_W = BATCH // NUM_WORKERS

CH_TC = 8
CH_W = CH_TC * 128
N_SLABS = EMBED_DIM // 8

U_FULL_TC = NUM_USERS // 128
U_TAIL = NUM_USERS - U_FULL_TC * 128
I_FULL_TC = NUM_ITEMS // 128
I_TAIL = NUM_ITEMS - I_FULL_TC * 128

U_TC_BASE, U_TC_EXTRA = divmod(U_FULL_TC, NUM_WORKERS)
I_TC_BASE, I_TC_EXTRA = divmod(I_FULL_TC, NUM_WORKERS)
U_TRIPS = -(-(U_TC_BASE + 1) // CH_TC)
I_TRIPS = -(-(I_TC_BASE + 1) // CH_TC)

N_GROUPS = BATCH // LANES
CAP = 2048
PAD_ROWS = LANES
EX_WORDS = (BATCH + PAD_ROWS) * EMBED_DIM


def _extract_body(uidx_hbm, iidx_hbm, uT_hbm, iT_hbm, utail_hbm, itail_hbm,
                  uex, iex, idx_all, bpos, bid, chpos, chloc, rowstage,
                  tailv, s0, s1, s2, s3, sem_in, sem_out):
    wid = lax.axis_index("s") * NUM_CORES + lax.axis_index("c")
    slabs = (s0, s1, s2, s3)
    iota = lax.iota(jnp.int32, LANES)
    iota32 = iota * EMBED_DIM

    def table_pass(idx_hbm, tab_hbm, tail_hbm, ex, base_tc, extra_tc,
                   full_tc, n_ids, tail_w, trips):
        ntc = base_tc + jnp.where(wid < extra_tc, 1, 0)
        start_tc = base_tc * wid + jnp.minimum(wid, extra_tc)
        lo = start_tc * 128
        hi = jnp.where(wid == NUM_WORKERS - 1, n_ids, (start_tc + ntc) * 128)

        pltpu.sync_copy(idx_hbm, idx_all)

        def bin_body(g, count):
            vec = idx_all[pl.ds(pl.multiple_of(g * LANES, LANES), LANES)]
            m = (vec >= lo) & (vec < hi)
            cum = plsc.cumsum(m.astype(jnp.int32))
            slots = count + cum - 1
            plsc.store_scatter(bid, [slots], vec, mask=m)
            plsc.store_scatter(bpos, [slots], g * LANES + iota, mask=m)
            return count + cum[LANES - 1]

        nbin = lax.fori_loop(0, N_GROUPS, bin_body, jnp.int32(0),
                             unroll=False)
        nsgroups = (nbin + LANES - 1) // LANES

        def process_range(rlo, rhi, gather_fn):
            def compact_cond(kc):
                k, cc = kc
                return (k < nsgroups) & (cc <= CAP - LANES)

            def compact_body(kc):
                k, cc = kc
                o = pl.multiple_of(k * LANES, LANES)
                ids = bid[pl.ds(o, LANES)]
                poss = bpos[pl.ds(o, LANES)]
                m = ((k * LANES + iota) < nbin) & (ids >= rlo) & (ids < rhi)
                cum = plsc.cumsum(m.astype(jnp.int32))
                slots = cc + cum - 1
                plsc.store_scatter(chloc, [slots], ids - rlo, mask=m)
                plsc.store_scatter(chpos, [slots], poss, mask=m)
                return k + 1, cc + cum[LANES - 1]

            def dense_body(q, carry):
                o = pl.multiple_of(q * LANES, LANES)
                loc = chloc[pl.ds(o, LANES)]
                pos = chpos[pl.ds(o, LANES)]
                for d in range(EMBED_DIM):
                    v = gather_fn(d, loc)
                    plsc.store_scatter(rowstage, [iota32 + d], v)
                outs = []
                for j in range(LANES):
                    pj = pos[j]
                    outs.append(pltpu.async_copy(
                        rowstage.at[pl.ds(j * EMBED_DIM, EMBED_DIM)],
                        ex.at[pl.ds(pl.multiple_of(pj * EMBED_DIM, EMBED_DIM),
                                    EMBED_DIM)],
                        sem_out))
                for ob in outs:
                    ob.wait()
                return carry

            def outer_body(k):
                k2, cc = lax.while_loop(compact_cond, compact_body,
                                        (k, jnp.int32(0)))
                plsc.store_scatter(chpos, [cc + iota], BATCH + iota)
                plsc.store_scatter(chloc, [cc + iota],
                                   jnp.zeros((LANES,), jnp.int32))
                lax.fori_loop(0, (cc + LANES - 1) // LANES, dense_body, 0,
                              unroll=False)
                return k2

            lax.while_loop(lambda k: k < nsgroups, outer_body, jnp.int32(0))

        def trip_body(c, carry):
            tc0 = start_tc + jnp.minimum(c * CH_TC, ntc - CH_TC)
            col0 = pl.multiple_of(tc0 * 128, 128)
            cps = []
            for tr in range(N_SLABS):
                cps.append(pltpu.async_copy(
                    tab_hbm.at[pl.ds(tr * 8, 8), pl.ds(col0, CH_W)],
                    slabs[tr], sem_in))
            for cp in cps:
                cp.wait()

            def gather_slab(d, loc):
                return plsc.load_gather(
                    slabs[d // 8],
                    [jnp.full((LANES,), d % 8, jnp.int32), loc])

            process_range(col0, col0 + CH_W, gather_slab)
            return carry

        lax.fori_loop(0, trips, trip_body, 0, unroll=False)

        @pl.when(wid == NUM_WORKERS - 1)
        def _():
            n = EMBED_DIM * tail_w
            pltpu.async_copy(tail_hbm, tailv.at[pl.ds(0, n)], sem_in).wait()

            def gather_tail(d, loc):
                return plsc.load_gather(tailv, [loc + d * tail_w])

            process_range(full_tc * 128, n_ids, gather_tail)

    table_pass(uidx_hbm, uT_hbm, utail_hbm, uex,
               U_TC_BASE, U_TC_EXTRA, U_FULL_TC, NUM_USERS, U_TAIL, U_TRIPS)
    table_pass(iidx_hbm, iT_hbm, itail_hbm, iex,
               I_TC_BASE, I_TC_EXTRA, I_FULL_TC, NUM_ITEMS, I_TAIL, I_TRIPS)


def _combine_body(uex_hbm, iex_hbm, out_hbm, uv, iv, out_v, sem):
    wid = lax.axis_index("s") * NUM_CORES + lax.axis_index("c")
    base = wid * B_PER_W
    n = B_PER_W * EMBED_DIM
    cu = pltpu.async_copy(uex_hbm.at[pl.ds(base * EMBED_DIM, n)], uv, sem)
    ci = pltpu.async_copy(iex_hbm.at[pl.ds(base * EMBED_DIM, n)], iv, sem)
    cu.wait()
    ci.wait()
    iota32 = lax.iota(jnp.int32, LANES) * EMBED_DIM

    def group(g, carry):
        g0 = g * B_PER_W
        acc = jnp.zeros((LANES,), jnp.float32)
        for d in range(EMBED_DIM):
            sel = g0 + iota32 + d
            acc = acc + (plsc.load_gather(uv, [sel])
                         * plsc.load_gather(iv, [sel]))
        out_v[pl.ds(pl.multiple_of(g * LANES, LANES), LANES)] = acc
        return carry

    lax.fori_loop(0, B_PER_W // LANES, group, 0, unroll=False)
    pltpu.sync_copy(out_v, out_hbm.at[pl.ds(base, B_PER_W)])


def kernel(user_indices, item_indices, user_emb, item_emb):
    mesh = plsc.VectorSubcoreMesh(
        core_axis_name="c", subcore_axis_name="s",
        num_cores=NUM_CORES, num_subcores=NUM_SUBCORES)

    extract = pl.kernel(
        _extract_body,
        out_type=(jax.ShapeDtypeStruct((EX_WORDS,), jnp.float32),
                  jax.ShapeDtypeStruct((EX_WORDS,), jnp.float32)),
        mesh=mesh,
        compiler_params=pltpu.CompilerParams(
            needs_layout_passes=False, use_tc_tiling_on_sc=True),
        scratch_types=[
            pltpu.VMEM((BATCH,), jnp.int32),
            pltpu.VMEM((BATCH,), jnp.int32),
            pltpu.VMEM((BATCH,), jnp.int32),
            pltpu.VMEM((CAP + LANES,), jnp.int32),
            pltpu.VMEM((CAP + LANES,), jnp.int32),
            pltpu.VMEM((LANES * EMBED_DIM,), jnp.float32),
            pltpu.VMEM((EMBED_DIM * I_TAIL,), jnp.float32),
            pltpu.VMEM((8, CH_W), jnp.float32),
            pltpu.VMEM((8, CH_W), jnp.float32),
            pltpu.VMEM((8, CH_W), jnp.float32),
            pltpu.VMEM((8, CH_W), jnp.float32),
            pltpu.SemaphoreType.DMA,
            pltpu.SemaphoreType.DMA,
        ],
    )

    combine = pl.kernel(
        _combine_body,
        out_type=jax.ShapeDtypeStruct((BATCH,), jnp.float32),
        mesh=mesh,
        compiler_params=pltpu.CompilerParams(
            needs_layout_passes=False, use_tc_tiling_on_sc=True),
        scratch_types=[
            pltpu.VMEM((B_PER_W * EMBED_DIM,), jnp.float32),
            pltpu.VMEM((B_PER_W * EMBED_DIM,), jnp.float32),
            pltpu.VMEM((B_PER_W,), jnp.float32),
            pltpu.SemaphoreType.DMA,
        ],
    )

    utail = user_emb[U_FULL_TC * 128:].T.reshape(-1)
    itail = item_emb[I_FULL_TC * 128:].T.reshape(-1)
    uex, iex = extract(user_indices.astype(jnp.int32),
                       item_indices.astype(jnp.int32),
                       user_emb.T, item_emb.T, utail, itail)
    return combine(uex, iex)


# speedup vs baseline: 14.1595x; 1.1855x over previous
import functools

import jax
import jax.numpy as jnp
from jax import lax
from jax.experimental import pallas as pl
from jax.experimental.pallas import tpu as pltpu
from jax.experimental.pallas import tpu_sc as plsc

NUM_CORES = 2
NUM_SUBCORES = 16
LANES = 16
NUM_WORKERS = NUM_CORES * NUM_SUBCORES

NUM_USERS = 100000
NUM_ITEMS = 1000000
BATCH = 16384
EMBED_DIM = 32
B_PER_W = BATCH // NUM_WORKERS

CH_TC = 8
CH_W = CH_TC * 128
N_SLABS = EMBED_DIM // 8

U_FULL_TC = NUM_USERS // 128
U_TAIL = NUM_USERS - U_FULL_TC * 128
I_FULL_TC = NUM_ITEMS // 128
I_TAIL = NUM_ITEMS - I_FULL_TC * 128

U_TC_BASE, U_TC_EXTRA = divmod(U_FULL_TC, NUM_WORKERS)
I_TC_BASE, I_TC_EXTRA = divmod(I_FULL_TC, NUM_WORKERS)
U_TRIPS = -(-(U_TC_BASE + 1) // CH_TC)
I_TRIPS = -(-(I_TC_BASE + 1) // CH_TC)

N_GROUPS = BATCH // LANES
CAP = 2048
PAD_ROWS = LANES
EX_WORDS = (BATCH + PAD_ROWS) * EMBED_DIM
RING = 8
ROW_WORDS = LANES * EMBED_DIM


def _extract_body(uidx_hbm, iidx_hbm, uT_hbm, iT_hbm, utail_hbm, itail_hbm,
                  uex, iex, idx_all, bpos, bid, chpos, chloc, rowstage,
                  tailv, s0, s1, s2, s3, sem_in, sem_out):
    wid = lax.axis_index("s") * NUM_CORES + lax.axis_index("c")
    slabs = (s0, s1, s2, s3)
    iota = lax.iota(jnp.int32, LANES)
    iota32 = iota * EMBED_DIM

    def table_pass(idx_hbm, tab_hbm, tail_hbm, ex, base_tc, extra_tc,
                   full_tc, n_ids, tail_w, trips):
        ntc = base_tc + jnp.where(wid < extra_tc, 1, 0)
        start_tc = base_tc * wid + jnp.minimum(wid, extra_tc)
        lo = start_tc * 128
        hi = jnp.where(wid == NUM_WORKERS - 1, n_ids, (start_tc + ntc) * 128)

        pltpu.sync_copy(idx_hbm, idx_all)

        def bin_body(g, count):
            vec = idx_all[pl.ds(pl.multiple_of(g * LANES, LANES), LANES)]
            m = (vec >= lo) & (vec < hi)
            cum = plsc.cumsum(m.astype(jnp.int32))
            slots = count + cum - 1
            plsc.store_scatter(bid, [slots], vec, mask=m)
            plsc.store_scatter(bpos, [slots], g * LANES + iota, mask=m)
            return count + cum[LANES - 1]

        nbin = lax.fori_loop(0, N_GROUPS, bin_body, jnp.int32(0),
                             unroll=4)
        nsgroups = (nbin + LANES - 1) // LANES

        def drain_rows():
            pltpu.make_async_copy(
                ex.at[pl.ds(0, ROW_WORDS)],
                rowstage.at[pl.ds(0, ROW_WORDS)], sem_out).wait()

        def process_range(rlo, rhi, gather_fn):
            def compact_cond(kc):
                k, cc = kc
                return (k < nsgroups) & (cc <= CAP - LANES)

            def compact_body(kc):
                k, cc = kc
                o = pl.multiple_of(k * LANES, LANES)
                ids = bid[pl.ds(o, LANES)]
                poss = bpos[pl.ds(o, LANES)]
                m = ((k * LANES + iota) < nbin) & (ids >= rlo) & (ids < rhi)
                cum = plsc.cumsum(m.astype(jnp.int32))
                slots = cc + cum - 1
                plsc.store_scatter(chloc, [slots], ids - rlo, mask=m)
                plsc.store_scatter(chpos, [slots], poss, mask=m)
                return k + 1, cc + cum[LANES - 1]

            def dense_body(q, carry):
                ro = (q % RING) * ROW_WORDS
                o = pl.multiple_of(q * LANES, LANES)
                loc = chloc[pl.ds(o, LANES)]
                pos = chpos[pl.ds(o, LANES)]

                @pl.when(q >= RING - 1)
                def _():
                    drain_rows()

                for d in range(EMBED_DIM):
                    v = gather_fn(d, loc)
                    plsc.store_scatter(rowstage, [ro + iota32 + d], v)
                for j in range(LANES):
                    pj = pos[j]
                    pltpu.async_copy(
                        rowstage.at[pl.ds(
                            pl.multiple_of(ro + j * EMBED_DIM, EMBED_DIM),
                            EMBED_DIM)],
                        ex.at[pl.ds(pl.multiple_of(pj * EMBED_DIM, EMBED_DIM),
                                    EMBED_DIM)],
                        sem_out)
                return carry

            def outer_body(k):
                k2, cc = lax.while_loop(compact_cond, compact_body,
                                        (k, jnp.int32(0)))

                @pl.when(cc > 0)
                def _():
                    plsc.store_scatter(chpos, [cc + iota], BATCH + iota)
                    plsc.store_scatter(chloc, [cc + iota],
                                       jnp.zeros((LANES,), jnp.int32))
                    nq = (cc + LANES - 1) // LANES
                    lax.fori_loop(0, nq, dense_body, 0, unroll=False)
                    for k3 in range(1, RING):
                        @pl.when(nq >= k3)
                        def _():
                            drain_rows()
                return k2

            lax.while_loop(lambda k: k < nsgroups, outer_body, jnp.int32(0))

        def fire_window(c):
            tc0 = start_tc + jnp.minimum(c * CH_TC, ntc - CH_TC)
            col0 = pl.multiple_of(tc0 * 128, 128)
            half = pl.multiple_of((c % 2) * CH_W, 128)
            for tr in range(N_SLABS):
                pltpu.async_copy(
                    tab_hbm.at[pl.ds(tr * 8, 8), pl.ds(col0, CH_W)],
                    slabs[tr].at[:, pl.ds(half, CH_W)], sem_in)

        def drain_window():
            for tr in range(N_SLABS):
                pltpu.make_async_copy(
                    tab_hbm.at[pl.ds(0, 8), pl.ds(0, CH_W)],
                    slabs[tr].at[:, pl.ds(0, CH_W)], sem_in).wait()

        fire_window(jnp.int32(0))

        def trip_body(c, carry):
            drain_window()

            @pl.when(c + 1 < trips)
            def _():
                fire_window(c + 1)

            tc0 = start_tc + jnp.minimum(c * CH_TC, ntc - CH_TC)
            col0 = pl.multiple_of(tc0 * 128, 128)
            half = (c % 2) * CH_W

            def gather_slab(d, loc):
                return plsc.load_gather(
                    slabs[d // 8],
                    [jnp.full((LANES,), d % 8, jnp.int32), half + loc])

            process_range(col0, col0 + CH_W, gather_slab)
            return carry

        lax.fori_loop(0, trips, trip_body, 0, unroll=False)

        @pl.when(wid == NUM_WORKERS - 1)
        def _():
            n = EMBED_DIM * tail_w
            pltpu.async_copy(tail_hbm, tailv.at[pl.ds(0, n)], sem_in).wait()

            def gather_tail(d, loc):
                return plsc.load_gather(tailv, [loc + d * tail_w])

            process_range(full_tc * 128, n_ids, gather_tail)

    table_pass(uidx_hbm, uT_hbm, utail_hbm, uex,
               U_TC_BASE, U_TC_EXTRA, U_FULL_TC, NUM_USERS, U_TAIL, U_TRIPS)
    table_pass(iidx_hbm, iT_hbm, itail_hbm, iex,
               I_TC_BASE, I_TC_EXTRA, I_FULL_TC, NUM_ITEMS, I_TAIL, I_TRIPS)


def _combine_body(uex_hbm, iex_hbm, out_hbm, uv, iv, out_v, sem):
    wid = lax.axis_index("s") * NUM_CORES + lax.axis_index("c")
    base = wid * B_PER_W
    n = B_PER_W * EMBED_DIM
    cu = pltpu.async_copy(uex_hbm.at[pl.ds(base * EMBED_DIM, n)], uv, sem)
    ci = pltpu.async_copy(iex_hbm.at[pl.ds(base * EMBED_DIM, n)], iv, sem)
    cu.wait()
    ci.wait()
    iota32 = lax.iota(jnp.int32, LANES) * EMBED_DIM

    def group(g, carry):
        g0 = g * B_PER_W
        acc = jnp.zeros((LANES,), jnp.float32)
        for d in range(EMBED_DIM):
            sel = g0 + iota32 + d
            acc = acc + (plsc.load_gather(uv, [sel])
                         * plsc.load_gather(iv, [sel]))
        out_v[pl.ds(pl.multiple_of(g * LANES, LANES), LANES)] = acc
        return carry

    lax.fori_loop(0, B_PER_W // LANES, group, 0, unroll=False)
    pltpu.sync_copy(out_v, out_hbm.at[pl.ds(base, B_PER_W)])


def kernel(user_indices, item_indices, user_emb, item_emb):
    mesh = plsc.VectorSubcoreMesh(
        core_axis_name="c", subcore_axis_name="s",
        num_cores=NUM_CORES, num_subcores=NUM_SUBCORES)

    extract = pl.kernel(
        _extract_body,
        out_type=(jax.ShapeDtypeStruct((EX_WORDS,), jnp.float32),
                  jax.ShapeDtypeStruct((EX_WORDS,), jnp.float32)),
        mesh=mesh,
        compiler_params=pltpu.CompilerParams(
            needs_layout_passes=False, use_tc_tiling_on_sc=True),
        scratch_types=[
            pltpu.VMEM((BATCH,), jnp.int32),
            pltpu.VMEM((BATCH,), jnp.int32),
            pltpu.VMEM((BATCH,), jnp.int32),
            pltpu.VMEM((CAP + LANES,), jnp.int32),
            pltpu.VMEM((CAP + LANES,), jnp.int32),
            pltpu.VMEM((RING * ROW_WORDS,), jnp.float32),
            pltpu.VMEM((EMBED_DIM * I_TAIL,), jnp.float32),
            pltpu.VMEM((8, 2 * CH_W), jnp.float32),
            pltpu.VMEM((8, 2 * CH_W), jnp.float32),
            pltpu.VMEM((8, 2 * CH_W), jnp.float32),
            pltpu.VMEM((8, 2 * CH_W), jnp.float32),
            pltpu.SemaphoreType.DMA,
            pltpu.SemaphoreType.DMA,
        ],
    )

    combine = pl.kernel(
        _combine_body,
        out_type=jax.ShapeDtypeStruct((BATCH,), jnp.float32),
        mesh=mesh,
        compiler_params=pltpu.CompilerParams(
            needs_layout_passes=False, use_tc_tiling_on_sc=True),
        scratch_types=[
            pltpu.VMEM((B_PER_W * EMBED_DIM,), jnp.float32),
            pltpu.VMEM((B_PER_W * EMBED_DIM,), jnp.float32),
            pltpu.VMEM((B_PER_W,), jnp.float32),
            pltpu.SemaphoreType.DMA,
        ],
    )

    utail = user_emb[U_FULL_TC * 128:].T.reshape(-1)
    itail = item_emb[I_FULL_TC * 128:].T.reshape(-1)
    uex, iex = extract(user_indices.astype(jnp.int32),
                       item_indices.astype(jnp.int32),
                       user_emb.T, item_emb.T, utail, itail)
    return combine(uex, iex)
